# baseline (speedup 1.0000x reference)
import functools

import jax
import jax.numpy as jnp
from jax import lax
from jax.experimental import pallas as pl
from jax.experimental.pallas import tpu as pltpu
from jax.experimental.pallas import tpu_sc as plsc

N = 10000
E = 320000
D = 128
NB = 16

W = 128
EPAD = 327680
NBUF = 2
AWIN = 120
BWIN = 40
WS = 128
NROWS = 10112
RPS = NROWS // 16

_vector_mesh = plsc.VectorSubcoreMesh(
    core_axis_name="core", subcore_axis_name="subcore")


def _sc_gather_add(p1f, idx_i, idx_j):
  @functools.partial(
      pl.kernel,
      out_type=jax.ShapeDtypeStruct((EPAD, D), jnp.float32),
      mesh=_vector_mesh,
      scratch_types=[
          pltpu.VMEM((AWIN, W), jnp.int32),
          pltpu.VMEM((AWIN, W), jnp.int32),
          pltpu.VMEM((NBUF, W, D), jnp.float32),
          pltpu.VMEM((NBUF, W, D), jnp.float32),
          pltpu.VMEM((W, D), jnp.float32),
          pltpu.SemaphoreType.DMA,
          pltpu.SemaphoreType.DMA,
          pltpu.SemaphoreType.DMA,
      ],
  )
  def k(p1_hbm, ii_hbm, jj_hbm, o_hbm, ii_v, jj_v, bi, bj, ob,
        gsem0, gsem1, osem):
    cid = lax.axis_index("core")
    sid = lax.axis_index("subcore")
    gsem = (gsem0, gsem1)
    wpw = jnp.where(cid == 0, AWIN, BWIN)
    base_win = jnp.where(cid == 0, sid * AWIN, 16 * AWIN + sid * BWIN)

    pltpu.sync_copy(ii_hbm.at[pl.ds(base_win, AWIN)], ii_v)
    pltpu.sync_copy(jj_hbm.at[pl.ds(base_win, AWIN)], jj_v)

    def start_gather(g, b):
      pltpu.async_copy(p1_hbm.at[ii_v.at[g]], bi.at[b], gsem[b])
      pltpu.async_copy(p1_hbm.at[jj_v.at[g]], bj.at[b], gsem[b])

    def wait_gather(g, b):
      pltpu.make_async_copy(p1_hbm.at[ii_v.at[g]], bi.at[b], gsem[b]).wait()
      pltpu.make_async_copy(p1_hbm.at[jj_v.at[g]], bj.at[b], gsem[b]).wait()

    def wait_out(g):
      pltpu.make_async_copy(
          ob, o_hbm.at[pl.ds((base_win + g) * W, W)], osem).wait()

    for b in range(NBUF):
      start_gather(b, b)

    @pl.loop(0, wpw, step=NBUF)
    def _(G):
      for b in range(NBUF):
        g = G + b
        wait_gather(g, b)

        @pl.when(g >= 1)
        def _():
          wait_out(g - 1)

        @pl.loop(0, W)
        def _(r):
          for c in range(0, D, 16):
            s = (r, pl.ds(c, 16))
            ob.at[s][...] = bi.at[b].at[s][...] + bj.at[b].at[s][...]

        @pl.when(g + NBUF < wpw)
        def _():
          start_gather(g + NBUF, b)

        pltpu.async_copy(
            ob, o_hbm.at[pl.ds((base_win + g) * W, W)], osem)

    wait_out(wpw - 1)

  return k(p1f, idx_i, idx_j)


def _sc_scatter_add(i1, scat_idx, zeros_init):
  @functools.partial(
      pl.kernel,
      out_type=jax.ShapeDtypeStruct((2, NROWS, D), jnp.float32),
      mesh=_vector_mesh,
      scratch_types=[pltpu.VMEM_SHARED((NROWS, D), jnp.float32)],
  )
  def k(i1_hbm, idx_hbm, z_hbm, o_hbm, acc_sh):
    cid = lax.axis_index("core")
    sid = lax.axis_index("subcore")
    half = EPAD // WS // 2

    pltpu.sync_copy(z_hbm.at[pl.ds(sid * RPS, RPS)],
                    acc_sh.at[pl.ds(sid * RPS, RPS)])
    plsc.subcore_barrier()

    def body(x_vmem, i_vmem):
      pltpu.sync_copy(x_vmem, acc_sh.at[i_vmem.at[0]], add=True)

    pltpu.emit_pipeline(
        body,
        grid=(half,),
        in_specs=[
            pl.BlockSpec((WS, D), lambda i: (i + cid * half, 0)),
            pl.BlockSpec((1, WS), lambda i: (0, i + cid * half)),
        ],
        out_specs=[],
        core_axis_name="subcore",
        dimension_semantics=(pltpu.PARALLEL,),
    )(i1_hbm, idx_hbm)

    plsc.subcore_barrier()
    pltpu.sync_copy(acc_sh.at[pl.ds(sid * RPS, RPS)],
                    o_hbm.at[cid, pl.ds(sid * RPS, RPS)])

  return k(i1, scat_idx, zeros_init)


def _tc_edge_mlp(x, basis, wpi_perm, wii):
  B = 512

  def body(x_ref, b_ref, wpi_ref, wii_ref, o_ref):
    xv = x_ref[...].astype(jnp.bfloat16)
    acc = jnp.zeros((B, D), jnp.float32)
    for b in range(NB):
      h = lax.dot_general(
          xv, wpi_ref[:, b * D:(b + 1) * D],
          (((1,), (0,)), ((), ())), preferred_element_type=jnp.float32)
      acc = acc + jnp.tanh(h) * b_ref[:, b:b + 1]
    o_ref[...] = jnp.tanh(lax.dot_general(
        acc.astype(jnp.bfloat16), wii_ref[...],
        (((1,), (0,)), ((), ())), preferred_element_type=jnp.float32))

  return pl.pallas_call(
      body,
      grid=(EPAD // B,),
      in_specs=[
          pl.BlockSpec((B, D), lambda i: (i, 0)),
          pl.BlockSpec((B, NB), lambda i: (i, 0)),
          pl.BlockSpec((D, D * NB), lambda i: (0, 0)),
          pl.BlockSpec((D, D), lambda i: (0, 0)),
      ],
      out_specs=pl.BlockSpec((B, D), lambda i: (i, 0)),
      out_shape=jax.ShapeDtypeStruct((EPAD, D), jnp.float32),
  )(x, basis, wpi_perm, wii)


def _tc_node_mlp(acc2, wpp, bpp):
  B = 400

  def body(a_ref, wpp_ref, bpp_ref, o_ref):
    a = a_ref[0] + a_ref[1]
    o_ref[...] = jnp.tanh(lax.dot_general(
        a, wpp_ref[...],
        (((1,), (0,)), ((), ())), preferred_element_type=jnp.float32)
        + bpp_ref[...])

  return pl.pallas_call(
      body,
      grid=(N // B,),
      in_specs=[
          pl.BlockSpec((2, B, D), lambda i: (0, i, 0)),
          pl.BlockSpec((D, D), lambda i: (0, 0)),
          pl.BlockSpec((1, D), lambda i: (0, 0)),
      ],
      out_specs=pl.BlockSpec((B, D), lambda i: (i, 0)),
      out_shape=jax.ShapeDtypeStruct((N, D), jnp.float32),
  )(acc2, wpp, bpp)


def kernel(p1, pair_i, pair_j, basis, W_pi, W_ii, W_pp, b_pp):
  p1f = p1.reshape(N, D)
  pad = EPAD - E
  idx_rows = 16 * AWIN + 15 * BWIN + AWIN
  ii = jnp.concatenate(
      [pair_i, jnp.zeros((idx_rows * W - E,), jnp.int32)]).reshape(
          idx_rows, W)
  jj = jnp.concatenate(
      [pair_j, jnp.zeros((idx_rows * W - E,), jnp.int32)]).reshape(
          idx_rows, W)
  scat = jnp.concatenate(
      [pair_i, jnp.full((pad,), N, jnp.int32)]).reshape(1, EPAD)
  basis_pad = jnp.concatenate(
      [basis, jnp.zeros((pad, NB), jnp.float32)], axis=0)
  wpi_perm = W_pi.reshape(D, D, NB).transpose(0, 2, 1).reshape(
      D, D * NB).astype(jnp.bfloat16)
  wii_b = W_ii.astype(jnp.bfloat16)
  zeros_init = jnp.zeros((NROWS, D), jnp.float32)

  x = _sc_gather_add(p1f, ii, jj)
  i1 = _tc_edge_mlp(x, basis_pad, wpi_perm, wii_b)
  acc2 = _sc_scatter_add(i1, scat, zeros_init)
  p1_new = _tc_node_mlp(acc2, W_pp, b_pp.reshape(1, D))
  return (p1_new.reshape(N, 1, D), i1[:E].reshape(E, 1, D))


# speedup vs baseline: 1.0345x; 1.0345x over previous
import functools

import jax
import jax.numpy as jnp
from jax import lax
from jax.experimental import pallas as pl
from jax.experimental.pallas import tpu as pltpu
from jax.experimental.pallas import tpu_sc as plsc

N = 10000
E = 320000
D = 128
NB = 16

W = 128
EPAD = 327680
NBUF = 2
AWIN = 152
BWIN = 8
WS = 128
NROWS = 10112
RPS = NROWS // 16

_vector_mesh = plsc.VectorSubcoreMesh(
    core_axis_name="core", subcore_axis_name="subcore")


def _sc_gather_add(p1f, idx_i, idx_j):
  @functools.partial(
      pl.kernel,
      out_type=jax.ShapeDtypeStruct((EPAD, D), jnp.float32),
      mesh=_vector_mesh,
      scratch_types=[
          pltpu.VMEM((AWIN, W), jnp.int32),
          pltpu.VMEM((AWIN, W), jnp.int32),
          pltpu.VMEM((NBUF, W, D), jnp.float32),
          pltpu.VMEM((NBUF, W, D), jnp.float32),
          pltpu.VMEM((W, D), jnp.float32),
          pltpu.SemaphoreType.DMA,
          pltpu.SemaphoreType.DMA,
          pltpu.SemaphoreType.DMA,
      ],
  )
  def k(p1_hbm, ii_hbm, jj_hbm, o_hbm, ii_v, jj_v, bi, bj, ob,
        gsem0, gsem1, osem):
    cid = lax.axis_index("core")
    sid = lax.axis_index("subcore")
    gsem = (gsem0, gsem1)
    wpw = jnp.where(cid == 0, AWIN, BWIN)
    base_win = jnp.where(cid == 0, sid * AWIN, 16 * AWIN + sid * BWIN)

    pltpu.sync_copy(ii_hbm.at[pl.ds(base_win, AWIN)], ii_v)
    pltpu.sync_copy(jj_hbm.at[pl.ds(base_win, AWIN)], jj_v)

    def start_gather(g, b):
      pltpu.async_copy(p1_hbm.at[ii_v.at[g]], bi.at[b], gsem[b])
      pltpu.async_copy(p1_hbm.at[jj_v.at[g]], bj.at[b], gsem[b])

    def wait_gather(g, b):
      pltpu.make_async_copy(p1_hbm.at[ii_v.at[g]], bi.at[b], gsem[b]).wait()
      pltpu.make_async_copy(p1_hbm.at[jj_v.at[g]], bj.at[b], gsem[b]).wait()

    def wait_out(g):
      pltpu.make_async_copy(
          ob, o_hbm.at[pl.ds((base_win + g) * W, W)], osem).wait()

    for b in range(NBUF):
      start_gather(b, b)

    @pl.loop(0, wpw, step=NBUF)
    def _(G):
      for b in range(NBUF):
        g = G + b
        wait_gather(g, b)

        @pl.when(g >= 1)
        def _():
          wait_out(g - 1)

        @pl.loop(0, W)
        def _(r):
          for c in range(0, D, 16):
            s = (r, pl.ds(c, 16))
            ob.at[s][...] = bi.at[b].at[s][...] + bj.at[b].at[s][...]

        @pl.when(g + NBUF < wpw)
        def _():
          start_gather(g + NBUF, b)

        pltpu.async_copy(
            ob, o_hbm.at[pl.ds((base_win + g) * W, W)], osem)

    wait_out(wpw - 1)

  return k(p1f, idx_i, idx_j)


def _sc_scatter_add(i1, scat_idx, zeros_init):
  @functools.partial(
      pl.kernel,
      out_type=jax.ShapeDtypeStruct((2, NROWS, D), jnp.float32),
      mesh=_vector_mesh,
      scratch_types=[pltpu.VMEM_SHARED((NROWS, D), jnp.float32)],
  )
  def k(i1_hbm, idx_hbm, z_hbm, o_hbm, acc_sh):
    cid = lax.axis_index("core")
    sid = lax.axis_index("subcore")
    half = EPAD // WS // 2

    pltpu.sync_copy(z_hbm.at[pl.ds(sid * RPS, RPS)],
                    acc_sh.at[pl.ds(sid * RPS, RPS)])
    plsc.subcore_barrier()

    def body(x_vmem, i_vmem):
      pltpu.sync_copy(x_vmem, acc_sh.at[i_vmem.at[0]], add=True)

    pltpu.emit_pipeline(
        body,
        grid=(half,),
        in_specs=[
            pl.BlockSpec((WS, D), lambda i: (i + cid * half, 0)),
            pl.BlockSpec((1, WS), lambda i: (0, i + cid * half)),
        ],
        out_specs=[],
        core_axis_name="subcore",
        dimension_semantics=(pltpu.PARALLEL,),
    )(i1_hbm, idx_hbm)

    plsc.subcore_barrier()
    pltpu.sync_copy(acc_sh.at[pl.ds(sid * RPS, RPS)],
                    o_hbm.at[cid, pl.ds(sid * RPS, RPS)])

  return k(i1, scat_idx, zeros_init)


def _tc_edge_mlp(x, basis, wpi_perm, wii):
  B = 512

  def body(x_ref, b_ref, wpi_ref, wii_ref, o_ref):
    xv = x_ref[...].astype(jnp.bfloat16)
    acc = jnp.zeros((B, D), jnp.float32)
    for b in range(NB):
      h = lax.dot_general(
          xv, wpi_ref[:, b * D:(b + 1) * D],
          (((1,), (0,)), ((), ())), preferred_element_type=jnp.float32)
      acc = acc + jnp.tanh(h) * b_ref[:, b:b + 1]
    o_ref[...] = jnp.tanh(lax.dot_general(
        acc.astype(jnp.bfloat16), wii_ref[...],
        (((1,), (0,)), ((), ())), preferred_element_type=jnp.float32))

  return pl.pallas_call(
      body,
      grid=(EPAD // B,),
      in_specs=[
          pl.BlockSpec((B, D), lambda i: (i, 0)),
          pl.BlockSpec((B, NB), lambda i: (i, 0)),
          pl.BlockSpec((D, D * NB), lambda i: (0, 0)),
          pl.BlockSpec((D, D), lambda i: (0, 0)),
      ],
      out_specs=pl.BlockSpec((B, D), lambda i: (i, 0)),
      out_shape=jax.ShapeDtypeStruct((EPAD, D), jnp.float32),
  )(x, basis, wpi_perm, wii)


def _tc_node_mlp(acc2, wpp, bpp):
  B = 400

  def body(a_ref, wpp_ref, bpp_ref, o_ref):
    a = a_ref[0] + a_ref[1]
    o_ref[...] = jnp.tanh(lax.dot_general(
        a, wpp_ref[...],
        (((1,), (0,)), ((), ())), preferred_element_type=jnp.float32)
        + bpp_ref[...])

  return pl.pallas_call(
      body,
      grid=(N // B,),
      in_specs=[
          pl.BlockSpec((2, B, D), lambda i: (0, i, 0)),
          pl.BlockSpec((D, D), lambda i: (0, 0)),
          pl.BlockSpec((1, D), lambda i: (0, 0)),
      ],
      out_specs=pl.BlockSpec((B, D), lambda i: (i, 0)),
      out_shape=jax.ShapeDtypeStruct((N, D), jnp.float32),
  )(acc2, wpp, bpp)


def kernel(p1, pair_i, pair_j, basis, W_pi, W_ii, W_pp, b_pp):
  p1f = p1.reshape(N, D)
  pad = EPAD - E
  idx_rows = 16 * AWIN + 15 * BWIN + AWIN
  ii = jnp.concatenate(
      [pair_i, jnp.zeros((idx_rows * W - E,), jnp.int32)]).reshape(
          idx_rows, W)
  jj = jnp.concatenate(
      [pair_j, jnp.zeros((idx_rows * W - E,), jnp.int32)]).reshape(
          idx_rows, W)
  scat = jnp.concatenate(
      [pair_i, jnp.full((pad,), N, jnp.int32)]).reshape(1, EPAD)
  basis_pad = jnp.concatenate(
      [basis, jnp.zeros((pad, NB), jnp.float32)], axis=0)
  wpi_perm = W_pi.reshape(D, D, NB).transpose(0, 2, 1).reshape(
      D, D * NB).astype(jnp.bfloat16)
  wii_b = W_ii.astype(jnp.bfloat16)
  zeros_init = jnp.zeros((NROWS, D), jnp.float32)

  x = _sc_gather_add(p1f, ii, jj)
  i1 = _tc_edge_mlp(x, basis_pad, wpi_perm, wii_b)
  acc2 = _sc_scatter_add(i1, scat, zeros_init)
  p1_new = _tc_node_mlp(acc2, W_pp, b_pp.reshape(1, D))
  return (p1_new.reshape(N, 1, D), i1[:E].reshape(E, 1, D))
